# baseline (speedup 1.0000x reference)
import jax
import jax.numpy as jnp
from jax import lax
from jax.experimental import pallas as pl
from jax.experimental.pallas import tpu as pltpu
from jax.experimental.pallas import tpu_sc as plsc

N = 25000
D = 64
DH = 32
E = 400000
P = 100000
NC = 2
NS = 16

NPAD = 25088
EPAD = 409600
E_PER_TILE = EPAD // NS
ECHUNKS = E_PER_TILE // 1024
PPAD = 102400
P_PER_TILE = PPAD // NS
PCHUNKS = P_PER_TILE // 128

ROWS_PER_TILE = NPAD // NS
RB = 1000
NBLK = N // RB


def _dot(a, b):
    return jnp.dot(a, b, preferred_element_type=jnp.float32)


def _l2n(z):
    n = jnp.sqrt(jnp.sum(z * z, axis=1, keepdims=True))
    n = jnp.where(n == 0, jnp.ones_like(n), n)
    return z / n


def _tc_embed_body(huser_ref, hprod_ref, wue_ref, bue_ref, wie_ref, bie_ref,
                   wp_u_ref, wp_i_ref, hu_ref, hi_ref, m_ref):
    hu = _dot(huser_ref[...], wue_ref[...]) + bue_ref[...]
    hi = _dot(hprod_ref[...], wie_ref[...]) + bie_ref[...]
    hu_ref[...] = hu
    hi_ref[...] = hi
    m_ref[0] = jax.nn.relu(_dot(hu, wp_u_ref[...])).astype(jnp.bfloat16)
    m_ref[1] = jax.nn.relu(_dot(hi, wp_i_ref[...])).astype(jnp.bfloat16)


def _tc_embed(h_user, h_product, W_user_emb, b_user_emb, W_item_emb,
              b_item_emb, W_pre1_up, W_pre1_pu):
    full = lambda shape: pl.BlockSpec(shape, lambda i: tuple(0 for _ in shape))
    return pl.pallas_call(
        _tc_embed_body,
        grid=(NBLK,),
        in_specs=[
            pl.BlockSpec((RB, 128), lambda i: (i, 0)),
            pl.BlockSpec((RB, 128), lambda i: (i, 0)),
            full((128, D)), full((1, D)), full((128, D)), full((1, D)),
            full((D, D)), full((D, D)),
        ],
        out_specs=[
            pl.BlockSpec((RB, D), lambda i: (i, 0)),
            pl.BlockSpec((RB, D), lambda i: (i, 0)),
            pl.BlockSpec((2, RB, D), lambda i: (0, i, 0)),
        ],
        out_shape=[
            jax.ShapeDtypeStruct((N, D), jnp.float32),
            jax.ShapeDtypeStruct((N, D), jnp.float32),
            jax.ShapeDtypeStruct((2, N, D), jnp.bfloat16),
        ],
    )(h_user, h_product, W_user_emb, b_user_emb.reshape(1, D), W_item_emb,
      b_item_emb.reshape(1, D), W_pre1_up, W_pre1_pu)


def _make_tc_post(with_pre):
    def body(s_ref, cnt_ref, hu_ref, hi_ref, wsu_ref, wnu_ref,
             wsp_ref, wnp_ref, *rest):
        if with_pre:
            wp2u_ref, wp2p_ref, hu_out, hi_out, m_ref = rest
        else:
            hu_out, hi_out = rest[:2]
        s_i = s_ref[0].astype(jnp.float32)
        s_u = s_ref[1].astype(jnp.float32)
        c_i = cnt_ref[0][:, 0:1]
        c_u = cnt_ref[1][:, 0:1]
        neigh_i = jnp.where(c_i > 0, s_i / jnp.where(c_i > 0, c_i, 1.0), 0.0)
        neigh_u = jnp.where(c_u > 0, s_u / jnp.where(c_u > 0, c_u, 1.0), 0.0)
        hi_n = _l2n(jax.nn.relu(_dot(hi_ref[...], wsu_ref[...])
                                + _dot(neigh_i, wnu_ref[...])))
        hu_n = _l2n(jax.nn.relu(_dot(hu_ref[...], wsp_ref[...])
                                + _dot(neigh_u, wnp_ref[...])))
        hu_out[...] = hu_n
        hi_out[...] = hi_n
        if not with_pre:
            hub_out, hib_out = rest[2:]
            hub_out[...] = hu_n.astype(jnp.bfloat16)
            hib_out[...] = hi_n.astype(jnp.bfloat16)
        if with_pre:
            m_ref[0] = jax.nn.relu(_dot(hu_n, wp2u_ref[...])).astype(jnp.bfloat16)
            m_ref[1] = jax.nn.relu(_dot(hi_n, wp2p_ref[...])).astype(jnp.bfloat16)

    full = lambda shape: pl.BlockSpec(shape, lambda i: tuple(0 for _ in shape))
    in_specs = [
        pl.BlockSpec((2, RB, D), lambda i: (0, i, 0)),
        pl.BlockSpec((2, RB, 16), lambda i: (0, i, 0)),
        pl.BlockSpec((RB, D), lambda i: (i, 0)),
        pl.BlockSpec((RB, D), lambda i: (i, 0)),
        full((D, D)), full((D, D)), full((D, D)), full((D, D)),
    ]
    out_specs = [
        pl.BlockSpec((RB, D), lambda i: (i, 0)),
        pl.BlockSpec((RB, D), lambda i: (i, 0)),
    ]
    out_shape = [
        jax.ShapeDtypeStruct((N, D), jnp.float32),
        jax.ShapeDtypeStruct((N, D), jnp.float32),
    ]
    if with_pre:
        in_specs += [full((D, D)), full((D, D))]
        out_specs += [pl.BlockSpec((2, RB, D), lambda i: (0, i, 0))]
        out_shape += [jax.ShapeDtypeStruct((2, N, D), jnp.bfloat16)]
    else:
        out_specs += [pl.BlockSpec((RB, D), lambda i: (i, 0)),
                      pl.BlockSpec((RB, D), lambda i: (i, 0))]
        out_shape += [jax.ShapeDtypeStruct((N, D), jnp.bfloat16),
                      jax.ShapeDtypeStruct((N, D), jnp.bfloat16)]

    def call(*args):
        return pl.pallas_call(body, grid=(NBLK,), in_specs=in_specs,
                              out_specs=out_specs, out_shape=out_shape)(*args)
    return call


def _zero_rows(ref, nrows, width, dtype=jnp.float32):
    lanes = 32 if dtype == jnp.bfloat16 else 16
    z = jnp.zeros((lanes,), dtype)

    def row(r, _):
        for k in range(width // lanes):
            ref[r, pl.ds(k * lanes, lanes)] = z
        return 0

    lax.fori_loop(0, nrows, row, 0)


def _sc_mesh():
    return plsc.VectorSubcoreMesh(core_axis_name="c", subcore_axis_name="s",
                                  num_cores=NC, num_subcores=NS)


def _sc_params():
    return pltpu.CompilerParams(use_tc_tiling_on_sc=False,
                                needs_layout_passes=False)


def _segsum_body(m_all, comb, s_out, cidx, msg, zbuf, acc, sem):
    cid = lax.axis_index("c")
    sid = lax.axis_index("s")

    _zero_rows(zbuf, 112, D, jnp.bfloat16)
    row0 = sid * ROWS_PER_TILE
    g0 = sid * ECHUNKS

    def zloop(j, _):
        pltpu.sync_copy(zbuf, acc.at[pl.ds(row0 + j * 112, 112)])
        return 0
    lax.fori_loop(0, ROWS_PER_TILE // 112, zloop, 0)

    plsc.subcore_barrier()

    def chunk(j, _):
        pltpu.sync_copy(comb.at[cid, g0 + j], cidx)
        for q in range(8):
            pltpu.async_copy(m_all.at[cid].at[cidx.at[q]],
                             msg.at[pl.ds(q * 128, 128)], sem)
        for q in range(8):
            pltpu.make_async_copy(m_all.at[cid].at[cidx.at[q]],
                                  msg.at[pl.ds(q * 128, 128)], sem).wait()
        for q in range(8):
            pltpu.sync_copy(msg.at[pl.ds(q * 128, 128)],
                            acc.at[cidx.at[8 + q]], add=True)
        return 0
    lax.fori_loop(0, ECHUNKS, chunk, 0)

    plsc.subcore_barrier()

    def out_loop(j, _):
        rows = pl.ds(row0 + j * 112, 112)
        pltpu.sync_copy(acc.at[rows], zbuf)
        pltpu.sync_copy(zbuf, s_out.at[cid].at[rows])
        return 0
    lax.fori_loop(0, ROWS_PER_TILE // 112, out_loop, 0)


def _segsum_call(m_all, comb):
    return pl.kernel(
        _segsum_body,
        out_type=jax.ShapeDtypeStruct((NC, NPAD, D), jnp.bfloat16),
        mesh=_sc_mesh(),
        compiler_params=_sc_params(),
        scratch_types=[
            pltpu.VMEM((16, 128), jnp.int32),
            pltpu.VMEM((1024, D), jnp.bfloat16),
            pltpu.VMEM((112, D), jnp.bfloat16),
            pltpu.VMEM_SHARED((NPAD, D), jnp.bfloat16),
            pltpu.SemaphoreType.DMA,
        ],
    )(m_all, comb)


def _counts_body(comb, cnt_out, didx, ones_v, cbuf, cacc, sem):
    del sem
    cid = lax.axis_index("c")
    sid = lax.axis_index("s")

    _zero_rows(cbuf, 392, 16)
    one = jnp.ones((16,), jnp.float32)

    def orow(r, _):
        ones_v[r, pl.ds(0, 16)] = one
        return 0
    lax.fori_loop(0, 128, orow, 0)

    row0 = sid * ROWS_PER_TILE

    def czloop(j, _):
        pltpu.sync_copy(cbuf, cacc.at[pl.ds(row0 + j * 392, 392)])
        return 0
    lax.fori_loop(0, ROWS_PER_TILE // 392, czloop, 0)

    plsc.subcore_barrier()

    g0 = sid * ECHUNKS

    def chunk(j, _):
        pltpu.sync_copy(comb.at[cid, g0 + j, pl.ds(8, 8)], didx)
        for q in range(8):
            pltpu.sync_copy(ones_v, cacc.at[didx.at[q]], add=True)
        return 0
    lax.fori_loop(0, ECHUNKS, chunk, 0)

    plsc.subcore_barrier()

    def cout_loop(j, _):
        rows = pl.ds(row0 + j * 392, 392)
        pltpu.sync_copy(cacc.at[rows], cbuf)
        pltpu.sync_copy(cbuf, cnt_out.at[cid].at[rows])
        return 0
    lax.fori_loop(0, ROWS_PER_TILE // 392, cout_loop, 0)


def _counts_call(comb):
    return pl.kernel(
        _counts_body,
        out_type=jax.ShapeDtypeStruct((NC, NPAD, 16), jnp.float32),
        mesh=_sc_mesh(),
        compiler_params=_sc_params(),
        scratch_types=[
            pltpu.VMEM((8, 128), jnp.int32),
            pltpu.VMEM((128, 16), jnp.float32),
            pltpu.VMEM((392, 16), jnp.float32),
            pltpu.VMEM_SHARED((NPAD, 16), jnp.float32),
            pltpu.SemaphoreType.DMA,
        ],
    )(comb)


def _scores_body(hub, hib, pcomb, prod_out,
                 pidx, urows, vrows, prows, sem):
    cid = lax.axis_index("c")
    sid = lax.axis_index("s")
    idx_row0 = sid * (P_PER_TILE // 128)

    def chunk(j, _):
        rb = idx_row0 + j
        pltpu.sync_copy(pcomb.at[cid, pl.ds(rb, 1)], pidx)
        pltpu.async_copy(hub.at[pidx.at[0, 0]], urows, sem)
        pltpu.async_copy(hib.at[pidx.at[0, 1]], vrows, sem)
        pltpu.make_async_copy(hub.at[pidx.at[0, 0]], urows, sem).wait()
        pltpu.make_async_copy(hib.at[pidx.at[0, 1]], vrows, sem).wait()

        def row(p, _):
            prows[p, pl.ds(0, 32)] = (urows[p, pl.ds(0, 32)]
                                      * vrows[p, pl.ds(0, 32)])
            prows[p, pl.ds(32, 32)] = (urows[p, pl.ds(32, 32)]
                                       * vrows[p, pl.ds(32, 32)])
            return 0
        lax.fori_loop(0, 128, row, 0)
        pltpu.sync_copy(prows, prod_out.at[cid].at[pl.ds(rb * 128, 128)])
        return 0
    lax.fori_loop(0, P_PER_TILE // 128, chunk, 0)


def _scores_call(hub, hib, pcomb):
    return pl.kernel(
        _scores_body,
        out_type=jax.ShapeDtypeStruct((NC, PPAD, D), jnp.bfloat16),
        mesh=_sc_mesh(),
        compiler_params=_sc_params(),
        scratch_types=[
            pltpu.VMEM((1, 2, 128), jnp.int32),
            pltpu.VMEM((128, D), jnp.bfloat16),
            pltpu.VMEM((128, D), jnp.bfloat16),
            pltpu.VMEM((128, D), jnp.bfloat16),
            pltpu.SemaphoreType.DMA,
        ],
    )(hub, hib, pcomb)


SRB = 2048


def _rowsum_body(p_ref, o_ref):
    o_ref[...] = jnp.sum(p_ref[...].astype(jnp.float32), axis=2)


def _rowsum_call(prod):
    return pl.pallas_call(
        _rowsum_body,
        grid=(PPAD // SRB,),
        in_specs=[pl.BlockSpec((NC, SRB, D), lambda i: (0, i, 0))],
        out_specs=pl.BlockSpec((NC, SRB), lambda i: (0, i)),
        out_shape=jax.ShapeDtypeStruct((NC, PPAD), jnp.float32),
    )(prod)


def _pad_idx(a, value, total):
    a = a.astype(jnp.int32)
    return jnp.concatenate(
        [a, jnp.full((total - a.shape[0],), value, jnp.int32)])


_tc_post_pre = _make_tc_post(True)
_tc_post_final = _make_tc_post(False)


def kernel(h_user, h_product, edge_u, edge_i, pos_u, pos_i, neg_u, neg_i,
           W_user_emb, b_user_emb, W_item_emb, b_item_emb,
           W_pre1_up, W_neigh1_up, W_self1_up,
           W_pre1_pu, W_neigh1_pu, W_self1_pu,
           W_pre2_up, W_neigh2_up, W_self2_up,
           W_pre2_pu, W_neigh2_pu, W_self2_pu):
    hu, hi, m1 = _tc_embed(h_user, h_product, W_user_emb,
                           b_user_emb, W_item_emb, b_item_emb,
                           W_pre1_up, W_pre1_pu)

    def _comb2(a0, a1, b0, b1, apad, bpad, total):
        a = jnp.stack([_pad_idx(a0, apad, total),
                       _pad_idx(a1, apad, total)]).reshape(NC, -1, 8, 128)
        b = jnp.stack([_pad_idx(b0, bpad, total),
                       _pad_idx(b1, bpad, total)]).reshape(NC, -1, 8, 128)
        return jnp.concatenate([a, b], axis=2)

    comb = _comb2(edge_u, edge_i, edge_i, edge_u, 0, N, EPAD)

    cnt = _counts_call(comb)
    s1 = _segsum_call(m1, comb)
    hu1, hi1, m2 = _tc_post_pre(s1, cnt, hu, hi,
                                W_self1_up, W_neigh1_up,
                                W_self1_pu, W_neigh1_pu,
                                W_pre2_up, W_pre2_pu)
    s2 = _segsum_call(m2, comb)
    hu2, hi2, hub, hib = _tc_post_final(s2, cnt, hu1, hi1,
                                        W_self2_up, W_neigh2_up,
                                        W_self2_pu, W_neigh2_pu)

    pu = jnp.stack([_pad_idx(pos_u, 0, PPAD),
                    _pad_idx(neg_u, 0, PPAD)]).reshape(NC, -1, 1, 128)
    pi = jnp.stack([_pad_idx(pos_i, 0, PPAD),
                    _pad_idx(neg_i, 0, PPAD)]).reshape(NC, -1, 1, 128)
    pcomb = jnp.concatenate([pu, pi], axis=2)
    prod = _scores_call(hub, hib, pcomb)
    sc = _rowsum_call(prod)
    return hu2, hi2, sc[0, :P], sc[1, :P]


# speedup vs baseline: 1.1971x; 1.1971x over previous
import jax
import jax.numpy as jnp
from jax import lax
from jax.experimental import pallas as pl
from jax.experimental.pallas import tpu as pltpu
from jax.experimental.pallas import tpu_sc as plsc

N = 25000
D = 64
DH = 32
E = 400000
P = 100000
NC = 2
NS = 16

NPAD = 25088
EPAD = 409600
E_PER_TILE = EPAD // NS
ECHUNKS = E_PER_TILE // 1024
PPAD = 102400
P_PER_TILE = PPAD // NS
PCHUNKS = P_PER_TILE // 128

ROWS_PER_TILE = NPAD // NS
RB = 5000
NBLK = N // RB


def _dot(a, b):
    return jnp.dot(a, b, preferred_element_type=jnp.float32)


def _l2n(z):
    n = jnp.sqrt(jnp.sum(z * z, axis=1, keepdims=True))
    n = jnp.where(n == 0, jnp.ones_like(n), n)
    return z / n


def _tc_embed_body(huser_ref, hprod_ref, wue_ref, bue_ref, wie_ref, bie_ref,
                   wp_u_ref, wp_i_ref, hu_ref, hi_ref, m_ref):
    hu = _dot(huser_ref[...], wue_ref[...]) + bue_ref[...]
    hi = _dot(hprod_ref[...], wie_ref[...]) + bie_ref[...]
    hu_ref[...] = hu
    hi_ref[...] = hi
    m_ref[0] = jax.nn.relu(_dot(hu, wp_u_ref[...])).astype(jnp.bfloat16)
    m_ref[1] = jax.nn.relu(_dot(hi, wp_i_ref[...])).astype(jnp.bfloat16)


def _tc_embed(h_user, h_product, W_user_emb, b_user_emb, W_item_emb,
              b_item_emb, W_pre1_up, W_pre1_pu):
    full = lambda shape: pl.BlockSpec(shape, lambda i: tuple(0 for _ in shape))
    return pl.pallas_call(
        _tc_embed_body,
        grid=(NBLK,),
        in_specs=[
            pl.BlockSpec((RB, 128), lambda i: (i, 0)),
            pl.BlockSpec((RB, 128), lambda i: (i, 0)),
            full((128, D)), full((1, D)), full((128, D)), full((1, D)),
            full((D, D)), full((D, D)),
        ],
        out_specs=[
            pl.BlockSpec((RB, D), lambda i: (i, 0)),
            pl.BlockSpec((RB, D), lambda i: (i, 0)),
            pl.BlockSpec((2, RB, D), lambda i: (0, i, 0)),
        ],
        out_shape=[
            jax.ShapeDtypeStruct((N, D), jnp.float32),
            jax.ShapeDtypeStruct((N, D), jnp.float32),
            jax.ShapeDtypeStruct((2, N, D), jnp.bfloat16),
        ],
    )(h_user, h_product, W_user_emb, b_user_emb.reshape(1, D), W_item_emb,
      b_item_emb.reshape(1, D), W_pre1_up, W_pre1_pu)


def _make_tc_post(with_pre):
    def body(s_ref, cnt_ref, hu_ref, hi_ref, wsu_ref, wnu_ref,
             wsp_ref, wnp_ref, *rest):
        if with_pre:
            wp2u_ref, wp2p_ref, hu_out, hi_out, m_ref = rest
        else:
            hu_out, hi_out = rest[:2]
        s_i = s_ref[0].astype(jnp.float32)
        s_u = s_ref[1].astype(jnp.float32)
        c_i = cnt_ref[0][:, 0:1]
        c_u = cnt_ref[1][:, 0:1]
        neigh_i = jnp.where(c_i > 0, s_i / jnp.where(c_i > 0, c_i, 1.0), 0.0)
        neigh_u = jnp.where(c_u > 0, s_u / jnp.where(c_u > 0, c_u, 1.0), 0.0)
        hi_n = _l2n(jax.nn.relu(_dot(hi_ref[...], wsu_ref[...])
                                + _dot(neigh_i, wnu_ref[...])))
        hu_n = _l2n(jax.nn.relu(_dot(hu_ref[...], wsp_ref[...])
                                + _dot(neigh_u, wnp_ref[...])))
        hu_out[...] = hu_n
        hi_out[...] = hi_n
        if not with_pre:
            hub_out, hib_out = rest[2:]
            hub_out[...] = hu_n.astype(jnp.bfloat16)
            hib_out[...] = hi_n.astype(jnp.bfloat16)
        if with_pre:
            m_ref[0] = jax.nn.relu(_dot(hu_n, wp2u_ref[...])).astype(jnp.bfloat16)
            m_ref[1] = jax.nn.relu(_dot(hi_n, wp2p_ref[...])).astype(jnp.bfloat16)

    full = lambda shape: pl.BlockSpec(shape, lambda i: tuple(0 for _ in shape))
    in_specs = [
        pl.BlockSpec((2, RB, D), lambda i: (0, i, 0)),
        pl.BlockSpec((2, RB, 16), lambda i: (0, i, 0)),
        pl.BlockSpec((RB, D), lambda i: (i, 0)),
        pl.BlockSpec((RB, D), lambda i: (i, 0)),
        full((D, D)), full((D, D)), full((D, D)), full((D, D)),
    ]
    out_specs = [
        pl.BlockSpec((RB, D), lambda i: (i, 0)),
        pl.BlockSpec((RB, D), lambda i: (i, 0)),
    ]
    out_shape = [
        jax.ShapeDtypeStruct((N, D), jnp.float32),
        jax.ShapeDtypeStruct((N, D), jnp.float32),
    ]
    if with_pre:
        in_specs += [full((D, D)), full((D, D))]
        out_specs += [pl.BlockSpec((2, RB, D), lambda i: (0, i, 0))]
        out_shape += [jax.ShapeDtypeStruct((2, N, D), jnp.bfloat16)]
    else:
        out_specs += [pl.BlockSpec((RB, D), lambda i: (i, 0)),
                      pl.BlockSpec((RB, D), lambda i: (i, 0))]
        out_shape += [jax.ShapeDtypeStruct((N, D), jnp.bfloat16),
                      jax.ShapeDtypeStruct((N, D), jnp.bfloat16)]

    def call(*args):
        return pl.pallas_call(body, grid=(NBLK,), in_specs=in_specs,
                              out_specs=out_specs, out_shape=out_shape)(*args)
    return call


def _zero_rows(ref, nrows, width, dtype=jnp.float32):
    lanes = 32 if dtype == jnp.bfloat16 else 16
    z = jnp.zeros((lanes,), dtype)

    def row(r, _):
        for k in range(width // lanes):
            ref[r, pl.ds(k * lanes, lanes)] = z
        return 0

    lax.fori_loop(0, nrows, row, 0)


def _sc_mesh():
    return plsc.VectorSubcoreMesh(core_axis_name="c", subcore_axis_name="s",
                                  num_cores=NC, num_subcores=NS)


def _sc_params():
    return pltpu.CompilerParams(use_tc_tiling_on_sc=False,
                                needs_layout_passes=False)


def _segsum_body(m_all, comb, s_out, cidx, msg, zbuf, acc, sem):
    cid = lax.axis_index("c")
    sid = lax.axis_index("s")

    _zero_rows(zbuf, 112, D, jnp.bfloat16)
    row0 = sid * ROWS_PER_TILE
    g0 = sid * ECHUNKS

    def zloop(j, _):
        pltpu.sync_copy(zbuf, acc.at[pl.ds(row0 + j * 112, 112)])
        return 0
    lax.fori_loop(0, ROWS_PER_TILE // 112, zloop, 0)

    plsc.subcore_barrier()

    def chunk(j, _):
        pltpu.sync_copy(comb.at[cid, g0 + j], cidx)
        for q in range(8):
            pltpu.async_copy(m_all.at[cid].at[cidx.at[q]],
                             msg.at[pl.ds(q * 128, 128)], sem)
        for q in range(8):
            pltpu.make_async_copy(m_all.at[cid].at[cidx.at[q]],
                                  msg.at[pl.ds(q * 128, 128)], sem).wait()
        for q in range(8):
            pltpu.sync_copy(msg.at[pl.ds(q * 128, 128)],
                            acc.at[cidx.at[8 + q]], add=True)
        return 0
    lax.fori_loop(0, ECHUNKS, chunk, 0)

    plsc.subcore_barrier()

    def out_loop(j, _):
        rows = pl.ds(row0 + j * 112, 112)
        pltpu.sync_copy(acc.at[rows], zbuf)
        pltpu.sync_copy(zbuf, s_out.at[cid].at[rows])
        return 0
    lax.fori_loop(0, ROWS_PER_TILE // 112, out_loop, 0)


def _segsum_call(m_all, comb):
    return pl.kernel(
        _segsum_body,
        out_type=jax.ShapeDtypeStruct((NC, NPAD, D), jnp.bfloat16),
        mesh=_sc_mesh(),
        compiler_params=_sc_params(),
        scratch_types=[
            pltpu.VMEM((16, 128), jnp.int32),
            pltpu.VMEM((1024, D), jnp.bfloat16),
            pltpu.VMEM((112, D), jnp.bfloat16),
            pltpu.VMEM_SHARED((NPAD, D), jnp.bfloat16),
            pltpu.SemaphoreType.DMA,
        ],
    )(m_all, comb)


def _counts_body(comb, cnt_out, didx, ones_v, cbuf, cacc, sem):
    del sem
    cid = lax.axis_index("c")
    sid = lax.axis_index("s")

    _zero_rows(cbuf, 392, 16)
    one = jnp.ones((16,), jnp.float32)

    def orow(r, _):
        ones_v[r, pl.ds(0, 16)] = one
        return 0
    lax.fori_loop(0, 128, orow, 0)

    row0 = sid * ROWS_PER_TILE

    def czloop(j, _):
        pltpu.sync_copy(cbuf, cacc.at[pl.ds(row0 + j * 392, 392)])
        return 0
    lax.fori_loop(0, ROWS_PER_TILE // 392, czloop, 0)

    plsc.subcore_barrier()

    g0 = sid * ECHUNKS

    def chunk(j, _):
        pltpu.sync_copy(comb.at[cid, g0 + j, pl.ds(8, 8)], didx)
        for q in range(8):
            pltpu.sync_copy(ones_v, cacc.at[didx.at[q]], add=True)
        return 0
    lax.fori_loop(0, ECHUNKS, chunk, 0)

    plsc.subcore_barrier()

    def cout_loop(j, _):
        rows = pl.ds(row0 + j * 392, 392)
        pltpu.sync_copy(cacc.at[rows], cbuf)
        pltpu.sync_copy(cbuf, cnt_out.at[cid].at[rows])
        return 0
    lax.fori_loop(0, ROWS_PER_TILE // 392, cout_loop, 0)


def _counts_call(comb):
    return pl.kernel(
        _counts_body,
        out_type=jax.ShapeDtypeStruct((NC, NPAD, 16), jnp.float32),
        mesh=_sc_mesh(),
        compiler_params=_sc_params(),
        scratch_types=[
            pltpu.VMEM((8, 128), jnp.int32),
            pltpu.VMEM((128, 16), jnp.float32),
            pltpu.VMEM((392, 16), jnp.float32),
            pltpu.VMEM_SHARED((NPAD, 16), jnp.float32),
            pltpu.SemaphoreType.DMA,
        ],
    )(comb)


def _scores_body(hub, hib, pcomb, sc_out,
                 pidx, urows, vrows, tbuf, sbuf, sem):
    cid = lax.axis_index("c")
    sid = lax.axis_index("s")
    idx_row0 = sid * (P_PER_TILE // 128)
    lanes = lax.iota(jnp.int32, 16)

    def chunk(j, _):
        rb = idx_row0 + j
        pltpu.sync_copy(pcomb.at[cid, pl.ds(rb, 1)], pidx)
        pltpu.async_copy(hub.at[pidx.at[0, 0]], urows, sem)
        pltpu.async_copy(hib.at[pidx.at[0, 1]], vrows, sem)
        pltpu.make_async_copy(hub.at[pidx.at[0, 0]], urows, sem).wait()
        pltpu.make_async_copy(hib.at[pidx.at[0, 1]], vrows, sem).wait()

        def group(g, _):
            for p in range(16):
                pa = (urows[g * 16 + p, pl.ds(0, 32)]
                      * vrows[g * 16 + p, pl.ds(0, 32)])
                pb = (urows[g * 16 + p, pl.ds(32, 32)]
                      * vrows[g * 16 + p, pl.ds(32, 32)])
                a0, a1 = plsc.unpack(pa, format=plsc.PackFormat.INTERLEAVED)
                b0, b1 = plsc.unpack(pb, format=plsc.PackFormat.INTERLEAVED)
                acc = (a0 + a1) + (b0 + b1)
                plsc.store_scatter(tbuf, [lanes, jnp.full((16,), p, jnp.int32)],
                                   acc)
            tot = tbuf[0, pl.ds(0, 16)]
            for rr in range(1, 16):
                tot = tot + tbuf[rr, pl.ds(0, 16)]
            sbuf[pl.ds(g * 16, 16)] = tot
            return 0
        lax.fori_loop(0, 8, group, 0)
        pltpu.sync_copy(sbuf, sc_out.at[cid, pl.ds(rb * 128, 128)])
        return 0
    lax.fori_loop(0, P_PER_TILE // 128, chunk, 0)


def _scores_call(hub, hib, pcomb):
    return pl.kernel(
        _scores_body,
        out_type=jax.ShapeDtypeStruct((NC, PPAD), jnp.float32),
        mesh=_sc_mesh(),
        compiler_params=_sc_params(),
        scratch_types=[
            pltpu.VMEM((1, 2, 128), jnp.int32),
            pltpu.VMEM((128, D), jnp.bfloat16),
            pltpu.VMEM((128, D), jnp.bfloat16),
            pltpu.VMEM((16, 16), jnp.float32),
            pltpu.VMEM((128,), jnp.float32),
            pltpu.SemaphoreType.DMA,
        ],
    )(hub, hib, pcomb)


def _pad_idx(a, value, total):
    a = a.astype(jnp.int32)
    return jnp.concatenate(
        [a, jnp.full((total - a.shape[0],), value, jnp.int32)])


_tc_post_pre = _make_tc_post(True)
_tc_post_final = _make_tc_post(False)


def kernel(h_user, h_product, edge_u, edge_i, pos_u, pos_i, neg_u, neg_i,
           W_user_emb, b_user_emb, W_item_emb, b_item_emb,
           W_pre1_up, W_neigh1_up, W_self1_up,
           W_pre1_pu, W_neigh1_pu, W_self1_pu,
           W_pre2_up, W_neigh2_up, W_self2_up,
           W_pre2_pu, W_neigh2_pu, W_self2_pu):
    hu, hi, m1 = _tc_embed(h_user, h_product, W_user_emb,
                           b_user_emb, W_item_emb, b_item_emb,
                           W_pre1_up, W_pre1_pu)

    def _comb2(a0, a1, b0, b1, apad, bpad, total):
        a = jnp.stack([_pad_idx(a0, apad, total),
                       _pad_idx(a1, apad, total)]).reshape(NC, -1, 8, 128)
        b = jnp.stack([_pad_idx(b0, bpad, total),
                       _pad_idx(b1, bpad, total)]).reshape(NC, -1, 8, 128)
        return jnp.concatenate([a, b], axis=2)

    comb = _comb2(edge_u, edge_i, edge_i, edge_u, 0, N, EPAD)

    cnt = _counts_call(comb)
    s1 = _segsum_call(m1, comb)
    hu1, hi1, m2 = _tc_post_pre(s1, cnt, hu, hi,
                                W_self1_up, W_neigh1_up,
                                W_self1_pu, W_neigh1_pu,
                                W_pre2_up, W_pre2_pu)
    s2 = _segsum_call(m2, comb)
    hu2, hi2, hub, hib = _tc_post_final(s2, cnt, hu1, hi1,
                                        W_self2_up, W_neigh2_up,
                                        W_self2_pu, W_neigh2_pu)

    pu = jnp.stack([_pad_idx(pos_u, 0, PPAD),
                    _pad_idx(neg_u, 0, PPAD)]).reshape(NC, -1, 1, 128)
    pi = jnp.stack([_pad_idx(pos_i, 0, PPAD),
                    _pad_idx(neg_i, 0, PPAD)]).reshape(NC, -1, 1, 128)
    pcomb = jnp.concatenate([pu, pi], axis=2)
    sc = _scores_call(hub, hib, pcomb)
    return hu2, hi2, sc[0, :P], sc[1, :P]


# speedup vs baseline: 1.2320x; 1.0292x over previous
import jax
import jax.numpy as jnp
from jax import lax
from jax.experimental import pallas as pl
from jax.experimental.pallas import tpu as pltpu
from jax.experimental.pallas import tpu_sc as plsc

N = 25000
D = 64
DH = 32
E = 400000
P = 100000
NC = 2
NS = 16

NPAD = 25088
EPAD = 409600
E_PER_TILE = EPAD // NS
ECHUNKS = E_PER_TILE // 1024
PPAD = 102400
P_PER_TILE = PPAD // NS
PCHUNKS = P_PER_TILE // 128

ROWS_PER_TILE = NPAD // NS
RB = 5000
NBLK = N // RB


def _dot(a, b):
    return jnp.dot(a, b, preferred_element_type=jnp.float32)


def _l2n(z):
    n = jnp.sqrt(jnp.sum(z * z, axis=1, keepdims=True))
    n = jnp.where(n == 0, jnp.ones_like(n), n)
    return z / n


def _tc_embed_body(huser_ref, hprod_ref, wue_ref, bue_ref, wie_ref, bie_ref,
                   wp_u_ref, wp_i_ref, hu_ref, hi_ref, m_ref):
    hu = _dot(huser_ref[...], wue_ref[...]) + bue_ref[...]
    hi = _dot(hprod_ref[...], wie_ref[...]) + bie_ref[...]
    hu_ref[...] = hu
    hi_ref[...] = hi
    m_ref[0] = jax.nn.relu(_dot(hu, wp_u_ref[...])).astype(jnp.bfloat16)
    m_ref[1] = jax.nn.relu(_dot(hi, wp_i_ref[...])).astype(jnp.bfloat16)


def _tc_embed(h_user, h_product, W_user_emb, b_user_emb, W_item_emb,
              b_item_emb, W_pre1_up, W_pre1_pu):
    full = lambda shape: pl.BlockSpec(shape, lambda i: tuple(0 for _ in shape))
    return pl.pallas_call(
        _tc_embed_body,
        grid=(NBLK,),
        in_specs=[
            pl.BlockSpec((RB, 128), lambda i: (i, 0)),
            pl.BlockSpec((RB, 128), lambda i: (i, 0)),
            full((128, D)), full((1, D)), full((128, D)), full((1, D)),
            full((D, D)), full((D, D)),
        ],
        out_specs=[
            pl.BlockSpec((RB, D), lambda i: (i, 0)),
            pl.BlockSpec((RB, D), lambda i: (i, 0)),
            pl.BlockSpec((2, RB, D), lambda i: (0, i, 0)),
        ],
        out_shape=[
            jax.ShapeDtypeStruct((N, D), jnp.float32),
            jax.ShapeDtypeStruct((N, D), jnp.float32),
            jax.ShapeDtypeStruct((2, N, D), jnp.bfloat16),
        ],
    )(h_user, h_product, W_user_emb, b_user_emb.reshape(1, D), W_item_emb,
      b_item_emb.reshape(1, D), W_pre1_up, W_pre1_pu)


def _make_tc_post(with_pre):
    def body(s_ref, cnt_ref, hu_ref, hi_ref, wsu_ref, wnu_ref,
             wsp_ref, wnp_ref, *rest):
        if with_pre:
            wp2u_ref, wp2p_ref, hu_out, hi_out, m_ref = rest
        else:
            hu_out, hi_out = rest[:2]
        s_i = s_ref[0].astype(jnp.float32)
        s_u = s_ref[1].astype(jnp.float32)
        c_i = cnt_ref[0][:, 0:1]
        c_u = cnt_ref[1][:, 0:1]
        neigh_i = jnp.where(c_i > 0, s_i / jnp.where(c_i > 0, c_i, 1.0), 0.0)
        neigh_u = jnp.where(c_u > 0, s_u / jnp.where(c_u > 0, c_u, 1.0), 0.0)
        hi_n = _l2n(jax.nn.relu(_dot(hi_ref[...], wsu_ref[...])
                                + _dot(neigh_i, wnu_ref[...])))
        hu_n = _l2n(jax.nn.relu(_dot(hu_ref[...], wsp_ref[...])
                                + _dot(neigh_u, wnp_ref[...])))
        hu_out[...] = hu_n
        hi_out[...] = hi_n
        if not with_pre:
            hub_out, hib_out = rest[2:]
            hub_out[...] = hu_n.astype(jnp.bfloat16)
            hib_out[...] = hi_n.astype(jnp.bfloat16)
        if with_pre:
            m_ref[0] = jax.nn.relu(_dot(hu_n, wp2u_ref[...])).astype(jnp.bfloat16)
            m_ref[1] = jax.nn.relu(_dot(hi_n, wp2p_ref[...])).astype(jnp.bfloat16)

    full = lambda shape: pl.BlockSpec(shape, lambda i: tuple(0 for _ in shape))
    in_specs = [
        pl.BlockSpec((2, RB, D), lambda i: (0, i, 0)),
        pl.BlockSpec((2, RB, 16), lambda i: (0, i, 0)),
        pl.BlockSpec((RB, D), lambda i: (i, 0)),
        pl.BlockSpec((RB, D), lambda i: (i, 0)),
        full((D, D)), full((D, D)), full((D, D)), full((D, D)),
    ]
    out_specs = [
        pl.BlockSpec((RB, D), lambda i: (i, 0)),
        pl.BlockSpec((RB, D), lambda i: (i, 0)),
    ]
    out_shape = [
        jax.ShapeDtypeStruct((N, D), jnp.float32),
        jax.ShapeDtypeStruct((N, D), jnp.float32),
    ]
    if with_pre:
        in_specs += [full((D, D)), full((D, D))]
        out_specs += [pl.BlockSpec((2, RB, D), lambda i: (0, i, 0))]
        out_shape += [jax.ShapeDtypeStruct((2, N, D), jnp.bfloat16)]
    else:
        out_specs += [pl.BlockSpec((RB, D), lambda i: (i, 0)),
                      pl.BlockSpec((RB, D), lambda i: (i, 0))]
        out_shape += [jax.ShapeDtypeStruct((N, D), jnp.bfloat16),
                      jax.ShapeDtypeStruct((N, D), jnp.bfloat16)]

    def call(*args):
        return pl.pallas_call(body, grid=(NBLK,), in_specs=in_specs,
                              out_specs=out_specs, out_shape=out_shape)(*args)
    return call


def _zero_rows(ref, nrows, width, dtype=jnp.float32):
    lanes = 32 if dtype == jnp.bfloat16 else 16
    z = jnp.zeros((lanes,), dtype)

    def row(r, _):
        for k in range(width // lanes):
            ref[r, pl.ds(k * lanes, lanes)] = z
        return 0

    lax.fori_loop(0, nrows, row, 0)


def _sc_mesh():
    return plsc.VectorSubcoreMesh(core_axis_name="c", subcore_axis_name="s",
                                  num_cores=NC, num_subcores=NS)


def _sc_params():
    return pltpu.CompilerParams(use_tc_tiling_on_sc=False,
                                needs_layout_passes=False)


def _segsum_body(m_all, comb, s_out, cidx, msg, zbuf, acc, sem, isem):
    cid = lax.axis_index("c")
    sid = lax.axis_index("s")

    _zero_rows(zbuf, 112, D, jnp.bfloat16)
    row0 = sid * ROWS_PER_TILE
    g0 = sid * ECHUNKS

    def zloop(j, _):
        pltpu.sync_copy(zbuf, acc.at[pl.ds(row0 + j * 112, 112)])
        return 0
    lax.fori_loop(0, ROWS_PER_TILE // 112, zloop, 0)

    plsc.subcore_barrier()

    pltpu.sync_copy(comb.at[cid, pl.ds(g0, 1)], cidx.at[pl.ds(0, 1)])

    def chunk(j, _):
        b = lax.rem(j, 2)
        nb = 1 - b
        jn = lax.min(j + 1, ECHUNKS - 1)
        pltpu.async_copy(comb.at[cid, pl.ds(g0 + jn, 1)],
                         cidx.at[pl.ds(nb, 1)], isem)
        for q in range(8):
            pltpu.async_copy(m_all.at[cid].at[cidx.at[b].at[q]],
                             msg.at[pl.ds(q * 128, 128)], sem)
        for q in range(8):
            pltpu.make_async_copy(m_all.at[cid].at[cidx.at[b].at[q]],
                                  msg.at[pl.ds(q * 128, 128)], sem).wait()
        for q in range(8):
            pltpu.sync_copy(msg.at[pl.ds(q * 128, 128)],
                            acc.at[cidx.at[b].at[8 + q]], add=True)
        pltpu.make_async_copy(comb.at[cid, pl.ds(g0 + jn, 1)],
                              cidx.at[pl.ds(nb, 1)], isem).wait()
        return 0
    lax.fori_loop(0, ECHUNKS, chunk, 0)

    plsc.subcore_barrier()

    def out_loop(j, _):
        rows = pl.ds(row0 + j * 112, 112)
        pltpu.sync_copy(acc.at[rows], zbuf)
        pltpu.sync_copy(zbuf, s_out.at[cid].at[rows])
        return 0
    lax.fori_loop(0, ROWS_PER_TILE // 112, out_loop, 0)


def _segsum_call(m_all, comb):
    return pl.kernel(
        _segsum_body,
        out_type=jax.ShapeDtypeStruct((NC, NPAD, D), jnp.bfloat16),
        mesh=_sc_mesh(),
        compiler_params=_sc_params(),
        scratch_types=[
            pltpu.VMEM((2, 16, 128), jnp.int32),
            pltpu.VMEM((1024, D), jnp.bfloat16),
            pltpu.VMEM((112, D), jnp.bfloat16),
            pltpu.VMEM_SHARED((NPAD, D), jnp.bfloat16),
            pltpu.SemaphoreType.DMA,
            pltpu.SemaphoreType.DMA,
        ],
    )(m_all, comb)


def _counts_body(comb, cnt_out, didx, ones_v, cbuf, cacc, sem):
    del sem
    cid = lax.axis_index("c")
    sid = lax.axis_index("s")

    _zero_rows(cbuf, 392, 16)
    one = jnp.ones((16,), jnp.float32)

    def orow(r, _):
        ones_v[r, pl.ds(0, 16)] = one
        return 0
    lax.fori_loop(0, 128, orow, 0)

    row0 = sid * ROWS_PER_TILE

    def czloop(j, _):
        pltpu.sync_copy(cbuf, cacc.at[pl.ds(row0 + j * 392, 392)])
        return 0
    lax.fori_loop(0, ROWS_PER_TILE // 392, czloop, 0)

    plsc.subcore_barrier()

    g0 = sid * ECHUNKS

    def chunk(j, _):
        pltpu.sync_copy(comb.at[cid, g0 + j, pl.ds(8, 8)], didx)
        for q in range(8):
            pltpu.sync_copy(ones_v, cacc.at[didx.at[q]], add=True)
        return 0
    lax.fori_loop(0, ECHUNKS, chunk, 0)

    plsc.subcore_barrier()

    def cout_loop(j, _):
        rows = pl.ds(row0 + j * 392, 392)
        pltpu.sync_copy(cacc.at[rows], cbuf)
        pltpu.sync_copy(cbuf, cnt_out.at[cid].at[rows])
        return 0
    lax.fori_loop(0, ROWS_PER_TILE // 392, cout_loop, 0)


def _counts_call(comb):
    return pl.kernel(
        _counts_body,
        out_type=jax.ShapeDtypeStruct((NC, NPAD, 16), jnp.float32),
        mesh=_sc_mesh(),
        compiler_params=_sc_params(),
        scratch_types=[
            pltpu.VMEM((8, 128), jnp.int32),
            pltpu.VMEM((128, 16), jnp.float32),
            pltpu.VMEM((392, 16), jnp.float32),
            pltpu.VMEM_SHARED((NPAD, 16), jnp.float32),
            pltpu.SemaphoreType.DMA,
        ],
    )(comb)


def _scores_body(hub, hib, pcomb, sc_out,
                 pidx, urows, vrows, tbuf, sbuf, sem):
    cid = lax.axis_index("c")
    sid = lax.axis_index("s")
    idx_row0 = sid * (P_PER_TILE // 128)
    lanes = lax.iota(jnp.int32, 16)

    def chunk(j, _):
        rb = idx_row0 + j
        pltpu.sync_copy(pcomb.at[cid, pl.ds(rb, 1)], pidx)
        pltpu.async_copy(hub.at[pidx.at[0, 0]], urows, sem)
        pltpu.async_copy(hib.at[pidx.at[0, 1]], vrows, sem)
        pltpu.make_async_copy(hub.at[pidx.at[0, 0]], urows, sem).wait()
        pltpu.make_async_copy(hib.at[pidx.at[0, 1]], vrows, sem).wait()

        def group(g, _):
            for p in range(16):
                pa = (urows[g * 16 + p, pl.ds(0, 32)]
                      * vrows[g * 16 + p, pl.ds(0, 32)]
                      + urows[g * 16 + p, pl.ds(32, 32)]
                      * vrows[g * 16 + p, pl.ds(32, 32)])
                a0, a1 = plsc.unpack(pa, format=plsc.PackFormat.INTERLEAVED)
                acc = a0 + a1
                plsc.store_scatter(tbuf, [lanes, jnp.full((16,), p, jnp.int32)],
                                   acc)
            tot = tbuf[0, pl.ds(0, 16)]
            for rr in range(1, 16):
                tot = tot + tbuf[rr, pl.ds(0, 16)]
            sbuf[pl.ds(g * 16, 16)] = tot
            return 0
        lax.fori_loop(0, 8, group, 0)
        pltpu.sync_copy(sbuf, sc_out.at[cid, pl.ds(rb * 128, 128)])
        return 0
    lax.fori_loop(0, P_PER_TILE // 128, chunk, 0)


def _scores_call(hub, hib, pcomb):
    return pl.kernel(
        _scores_body,
        out_type=jax.ShapeDtypeStruct((NC, PPAD), jnp.float32),
        mesh=_sc_mesh(),
        compiler_params=_sc_params(),
        scratch_types=[
            pltpu.VMEM((1, 2, 128), jnp.int32),
            pltpu.VMEM((128, D), jnp.bfloat16),
            pltpu.VMEM((128, D), jnp.bfloat16),
            pltpu.VMEM((16, 16), jnp.float32),
            pltpu.VMEM((128,), jnp.float32),
            pltpu.SemaphoreType.DMA,
        ],
    )(hub, hib, pcomb)


def _pad_idx(a, value, total):
    a = a.astype(jnp.int32)
    return jnp.concatenate(
        [a, jnp.full((total - a.shape[0],), value, jnp.int32)])


_tc_post_pre = _make_tc_post(True)
_tc_post_final = _make_tc_post(False)


def kernel(h_user, h_product, edge_u, edge_i, pos_u, pos_i, neg_u, neg_i,
           W_user_emb, b_user_emb, W_item_emb, b_item_emb,
           W_pre1_up, W_neigh1_up, W_self1_up,
           W_pre1_pu, W_neigh1_pu, W_self1_pu,
           W_pre2_up, W_neigh2_up, W_self2_up,
           W_pre2_pu, W_neigh2_pu, W_self2_pu):
    hu, hi, m1 = _tc_embed(h_user, h_product, W_user_emb,
                           b_user_emb, W_item_emb, b_item_emb,
                           W_pre1_up, W_pre1_pu)

    def _comb2(a0, a1, b0, b1, apad, bpad, total):
        a = jnp.stack([_pad_idx(a0, apad, total),
                       _pad_idx(a1, apad, total)]).reshape(NC, -1, 8, 128)
        b = jnp.stack([_pad_idx(b0, bpad, total),
                       _pad_idx(b1, bpad, total)]).reshape(NC, -1, 8, 128)
        return jnp.concatenate([a, b], axis=2)

    comb = _comb2(edge_u, edge_i, edge_i, edge_u, 0, N, EPAD)

    cnt = _counts_call(comb)
    s1 = _segsum_call(m1, comb)
    hu1, hi1, m2 = _tc_post_pre(s1, cnt, hu, hi,
                                W_self1_up, W_neigh1_up,
                                W_self1_pu, W_neigh1_pu,
                                W_pre2_up, W_pre2_pu)
    s2 = _segsum_call(m2, comb)
    hu2, hi2, hub, hib = _tc_post_final(s2, cnt, hu1, hi1,
                                        W_self2_up, W_neigh2_up,
                                        W_self2_pu, W_neigh2_pu)

    pu = jnp.stack([_pad_idx(pos_u, 0, PPAD),
                    _pad_idx(neg_u, 0, PPAD)]).reshape(NC, -1, 1, 128)
    pi = jnp.stack([_pad_idx(pos_i, 0, PPAD),
                    _pad_idx(neg_i, 0, PPAD)]).reshape(NC, -1, 1, 128)
    pcomb = jnp.concatenate([pu, pi], axis=2)
    sc = _scores_call(hub, hib, pcomb)
    return hu2, hi2, sc[0, :P], sc[1, :P]


# speedup vs baseline: 1.2903x; 1.0473x over previous
import jax
import jax.numpy as jnp
from jax import lax
from jax.experimental import pallas as pl
from jax.experimental.pallas import tpu as pltpu
from jax.experimental.pallas import tpu_sc as plsc

N = 25000
D = 64
DH = 32
E = 400000
P = 100000
NC = 2
NS = 16

NPAD = 25088
EPAD = 409600
E_PER_TILE = EPAD // NS
ECHUNKS = E_PER_TILE // 1024
PPAD = 102400
P_PER_TILE = PPAD // NS
PCHUNKS = P_PER_TILE // 128

ROWS_PER_TILE = NPAD // NS
RB = 5000
NBLK = N // RB


def _dot(a, b):
    return jnp.dot(a, b, preferred_element_type=jnp.float32)


def _l2n(z):
    n = jnp.sqrt(jnp.sum(z * z, axis=1, keepdims=True))
    n = jnp.where(n == 0, jnp.ones_like(n), n)
    return z / n


def _tc_embed_body(huser_ref, hprod_ref, wue_ref, bue_ref, wie_ref, bie_ref,
                   wp_u_ref, wp_i_ref, hu_ref, hi_ref, m_ref):
    hu = _dot(huser_ref[...], wue_ref[...]) + bue_ref[...]
    hi = _dot(hprod_ref[...], wie_ref[...]) + bie_ref[...]
    hu_ref[...] = hu
    hi_ref[...] = hi
    m_ref[0] = jax.nn.relu(_dot(hu, wp_u_ref[...])).astype(jnp.bfloat16)
    m_ref[1] = jax.nn.relu(_dot(hi, wp_i_ref[...])).astype(jnp.bfloat16)


def _tc_embed(h_user, h_product, W_user_emb, b_user_emb, W_item_emb,
              b_item_emb, W_pre1_up, W_pre1_pu):
    full = lambda shape: pl.BlockSpec(shape, lambda i: tuple(0 for _ in shape))
    return pl.pallas_call(
        _tc_embed_body,
        grid=(NBLK,),
        in_specs=[
            pl.BlockSpec((RB, 128), lambda i: (i, 0)),
            pl.BlockSpec((RB, 128), lambda i: (i, 0)),
            full((128, D)), full((1, D)), full((128, D)), full((1, D)),
            full((D, D)), full((D, D)),
        ],
        out_specs=[
            pl.BlockSpec((RB, D), lambda i: (i, 0)),
            pl.BlockSpec((RB, D), lambda i: (i, 0)),
            pl.BlockSpec((2, RB, D), lambda i: (0, i, 0)),
        ],
        out_shape=[
            jax.ShapeDtypeStruct((N, D), jnp.float32),
            jax.ShapeDtypeStruct((N, D), jnp.float32),
            jax.ShapeDtypeStruct((2, N, D), jnp.bfloat16),
        ],
    )(h_user, h_product, W_user_emb, b_user_emb.reshape(1, D), W_item_emb,
      b_item_emb.reshape(1, D), W_pre1_up, W_pre1_pu)


def _make_tc_post(with_pre):
    def body(s_ref, cnt_ref, hu_ref, hi_ref, wsu_ref, wnu_ref,
             wsp_ref, wnp_ref, *rest):
        if with_pre:
            wp2u_ref, wp2p_ref, hu_out, hi_out, m_ref = rest
        else:
            hu_out, hi_out = rest[:2]
        s_i = s_ref[0].astype(jnp.float32)
        s_u = s_ref[1].astype(jnp.float32)
        c_i = cnt_ref[0][:, 0:1]
        c_u = cnt_ref[1][:, 0:1]
        neigh_i = jnp.where(c_i > 0, s_i / jnp.where(c_i > 0, c_i, 1.0), 0.0)
        neigh_u = jnp.where(c_u > 0, s_u / jnp.where(c_u > 0, c_u, 1.0), 0.0)
        hi_n = _l2n(jax.nn.relu(_dot(hi_ref[...], wsu_ref[...])
                                + _dot(neigh_i, wnu_ref[...])))
        hu_n = _l2n(jax.nn.relu(_dot(hu_ref[...], wsp_ref[...])
                                + _dot(neigh_u, wnp_ref[...])))
        hu_out[...] = hu_n
        hi_out[...] = hi_n
        if not with_pre:
            hub_out, hib_out = rest[2:]
            hub_out[...] = hu_n.astype(jnp.bfloat16)
            hib_out[...] = hi_n.astype(jnp.bfloat16)
        if with_pre:
            m_ref[0] = jax.nn.relu(_dot(hu_n, wp2u_ref[...])).astype(jnp.bfloat16)
            m_ref[1] = jax.nn.relu(_dot(hi_n, wp2p_ref[...])).astype(jnp.bfloat16)

    full = lambda shape: pl.BlockSpec(shape, lambda i: tuple(0 for _ in shape))
    in_specs = [
        pl.BlockSpec((2, RB, D), lambda i: (0, i, 0)),
        pl.BlockSpec((2, RB, 16), lambda i: (0, i, 0)),
        pl.BlockSpec((RB, D), lambda i: (i, 0)),
        pl.BlockSpec((RB, D), lambda i: (i, 0)),
        full((D, D)), full((D, D)), full((D, D)), full((D, D)),
    ]
    out_specs = [
        pl.BlockSpec((RB, D), lambda i: (i, 0)),
        pl.BlockSpec((RB, D), lambda i: (i, 0)),
    ]
    out_shape = [
        jax.ShapeDtypeStruct((N, D), jnp.float32),
        jax.ShapeDtypeStruct((N, D), jnp.float32),
    ]
    if with_pre:
        in_specs += [full((D, D)), full((D, D))]
        out_specs += [pl.BlockSpec((2, RB, D), lambda i: (0, i, 0))]
        out_shape += [jax.ShapeDtypeStruct((2, N, D), jnp.bfloat16)]
    else:
        out_specs += [pl.BlockSpec((RB, D), lambda i: (i, 0)),
                      pl.BlockSpec((RB, D), lambda i: (i, 0))]
        out_shape += [jax.ShapeDtypeStruct((N, D), jnp.bfloat16),
                      jax.ShapeDtypeStruct((N, D), jnp.bfloat16)]

    def call(*args):
        return pl.pallas_call(body, grid=(NBLK,), in_specs=in_specs,
                              out_specs=out_specs, out_shape=out_shape)(*args)
    return call


def _zero_rows(ref, nrows, width, dtype=jnp.float32):
    lanes = 32 if dtype == jnp.bfloat16 else 16
    z = jnp.zeros((lanes,), dtype)

    def row(r, _):
        for k in range(width // lanes):
            ref[r, pl.ds(k * lanes, lanes)] = z
        return 0

    lax.fori_loop(0, nrows, row, 0)


def _sc_mesh():
    return plsc.VectorSubcoreMesh(core_axis_name="c", subcore_axis_name="s",
                                  num_cores=NC, num_subcores=NS)


def _sc_params():
    return pltpu.CompilerParams(use_tc_tiling_on_sc=False,
                                needs_layout_passes=False)


def _segsum_body(m_all, comb, s_out, cidx, msg, zbuf, acc, sem, isem):
    cid = lax.axis_index("c")
    sid = lax.axis_index("s")

    _zero_rows(zbuf, 112, D, jnp.bfloat16)
    row0 = sid * ROWS_PER_TILE
    g0 = sid * ECHUNKS

    def zloop(j, _):
        pltpu.sync_copy(zbuf, acc.at[pl.ds(row0 + j * 112, 112)])
        return 0
    lax.fori_loop(0, ROWS_PER_TILE // 112, zloop, 0)

    plsc.subcore_barrier()

    pltpu.sync_copy(comb.at[cid, pl.ds(g0, 1)], cidx.at[pl.ds(0, 1)])

    def chunk(j, _):
        b = lax.rem(j, 2)
        nb = 1 - b
        jn = lax.min(j + 1, ECHUNKS - 1)
        pltpu.async_copy(comb.at[cid, pl.ds(g0 + jn, 1)],
                         cidx.at[pl.ds(nb, 1)], isem)
        for q in range(8):
            pltpu.async_copy(m_all.at[cid].at[cidx.at[b].at[q]],
                             msg.at[pl.ds(q * 128, 128)], sem)
        for q in range(8):
            pltpu.make_async_copy(m_all.at[cid].at[cidx.at[b].at[q]],
                                  msg.at[pl.ds(q * 128, 128)], sem).wait()
        for q in range(8):
            pltpu.sync_copy(msg.at[pl.ds(q * 128, 128)],
                            acc.at[cidx.at[b].at[8 + q]], add=True)
        pltpu.make_async_copy(comb.at[cid, pl.ds(g0 + jn, 1)],
                              cidx.at[pl.ds(nb, 1)], isem).wait()
        return 0
    lax.fori_loop(0, ECHUNKS, chunk, 0)

    plsc.subcore_barrier()

    def out_loop(j, _):
        rows = pl.ds(row0 + j * 112, 112)
        pltpu.sync_copy(acc.at[rows], zbuf)
        pltpu.sync_copy(zbuf, s_out.at[cid].at[rows])
        return 0
    lax.fori_loop(0, ROWS_PER_TILE // 112, out_loop, 0)


def _segsum_call(m_all, comb):
    return pl.kernel(
        _segsum_body,
        out_type=jax.ShapeDtypeStruct((NC, NPAD, D), jnp.bfloat16),
        mesh=_sc_mesh(),
        compiler_params=_sc_params(),
        scratch_types=[
            pltpu.VMEM((2, 16, 128), jnp.int32),
            pltpu.VMEM((1024, D), jnp.bfloat16),
            pltpu.VMEM((112, D), jnp.bfloat16),
            pltpu.VMEM_SHARED((NPAD, D), jnp.bfloat16),
            pltpu.SemaphoreType.DMA,
            pltpu.SemaphoreType.DMA,
        ],
    )(m_all, comb)


def _counts_body(comb, cnt_out, didx, ones_v, cbuf, cacc, sem):
    del sem
    cid = lax.axis_index("c")
    sid = lax.axis_index("s")

    _zero_rows(cbuf, 392, 16)
    one = jnp.ones((16,), jnp.float32)

    def orow(r, _):
        ones_v[r, pl.ds(0, 16)] = one
        return 0
    lax.fori_loop(0, 128, orow, 0)

    row0 = sid * ROWS_PER_TILE

    def czloop(j, _):
        pltpu.sync_copy(cbuf, cacc.at[pl.ds(row0 + j * 392, 392)])
        return 0
    lax.fori_loop(0, ROWS_PER_TILE // 392, czloop, 0)

    plsc.subcore_barrier()

    g0 = sid * ECHUNKS

    def chunk(j, _):
        pltpu.sync_copy(comb.at[cid, g0 + j, pl.ds(8, 8)], didx)
        for q in range(8):
            pltpu.sync_copy(ones_v, cacc.at[didx.at[q]], add=True)
        return 0
    lax.fori_loop(0, ECHUNKS, chunk, 0)

    plsc.subcore_barrier()

    def cout_loop(j, _):
        rows = pl.ds(row0 + j * 392, 392)
        pltpu.sync_copy(cacc.at[rows], cbuf)
        pltpu.sync_copy(cbuf, cnt_out.at[cid].at[rows])
        return 0
    lax.fori_loop(0, ROWS_PER_TILE // 392, cout_loop, 0)


def _counts_call(comb):
    return pl.kernel(
        _counts_body,
        out_type=jax.ShapeDtypeStruct((NC, NPAD, 16), jnp.float32),
        mesh=_sc_mesh(),
        compiler_params=_sc_params(),
        scratch_types=[
            pltpu.VMEM((8, 128), jnp.int32),
            pltpu.VMEM((128, 16), jnp.float32),
            pltpu.VMEM((392, 16), jnp.float32),
            pltpu.VMEM_SHARED((NPAD, 16), jnp.float32),
            pltpu.SemaphoreType.DMA,
        ],
    )(comb)


def _scores_body(hub, hib, pcomb, sc_out,
                 pidx, urows, vrows, tbuf, sbuf, sem0, sem1):
    cid = lax.axis_index("c")
    sid = lax.axis_index("s")
    idx_row0 = sid * (P_PER_TILE // 128)
    lanes = lax.iota(jnp.int32, 16)
    nch = P_PER_TILE // 128
    sems = (sem0, sem1)

    def fire(b, rb):
        pltpu.sync_copy(pcomb.at[cid, pl.ds(rb, 1)], pidx.at[pl.ds(b, 1)])
        pltpu.async_copy(hub.at[pidx.at[b, 0]], urows.at[b], sems[b])
        pltpu.async_copy(hib.at[pidx.at[b, 1]], vrows.at[b], sems[b])

    def wait(b):
        pltpu.make_async_copy(hub.at[pidx.at[b, 0]], urows.at[b],
                              sems[b]).wait()
        pltpu.make_async_copy(hib.at[pidx.at[b, 1]], vrows.at[b],
                              sems[b]).wait()

    def compute(b, rb):
        def group(g, _):
            for p in range(16):
                pa = (urows[b, g * 16 + p, pl.ds(0, 32)]
                      * vrows[b, g * 16 + p, pl.ds(0, 32)]
                      + urows[b, g * 16 + p, pl.ds(32, 32)]
                      * vrows[b, g * 16 + p, pl.ds(32, 32)])
                a0, a1 = plsc.unpack(pa, format=plsc.PackFormat.INTERLEAVED)
                plsc.store_scatter(tbuf, [lanes, jnp.full((16,), p, jnp.int32)],
                                   a0 + a1)
            tot = tbuf[0, pl.ds(0, 16)]
            for rr in range(1, 16):
                tot = tot + tbuf[rr, pl.ds(0, 16)]
            sbuf[pl.ds(g * 16, 16)] = tot
            return 0
        lax.fori_loop(0, 8, group, 0)
        pltpu.sync_copy(sbuf, sc_out.at[cid, pl.ds(rb * 128, 128)])

    fire(0, idx_row0)

    def pair(g, _):
        j0 = g * 2
        for b in (0, 1):
            j = j0 + b
            wait(b)
            jn = lax.min(j + 1, nch - 1)
            fire(1 - b, idx_row0 + jn)
            compute(b, idx_row0 + j)
        return 0
    lax.fori_loop(0, nch // 2, pair, 0)

    wait(0)


def _scores_call(hub, hib, pcomb):
    return pl.kernel(
        _scores_body,
        out_type=jax.ShapeDtypeStruct((NC, PPAD), jnp.float32),
        mesh=_sc_mesh(),
        compiler_params=_sc_params(),
        scratch_types=[
            pltpu.VMEM((2, 2, 128), jnp.int32),
            pltpu.VMEM((2, 128, D), jnp.bfloat16),
            pltpu.VMEM((2, 128, D), jnp.bfloat16),
            pltpu.VMEM((16, 16), jnp.float32),
            pltpu.VMEM((128,), jnp.float32),
            pltpu.SemaphoreType.DMA,
            pltpu.SemaphoreType.DMA,
        ],
    )(hub, hib, pcomb)


def _pad_idx(a, value, total):
    a = a.astype(jnp.int32)
    return jnp.concatenate(
        [a, jnp.full((total - a.shape[0],), value, jnp.int32)])


_tc_post_pre = _make_tc_post(True)
_tc_post_final = _make_tc_post(False)


def kernel(h_user, h_product, edge_u, edge_i, pos_u, pos_i, neg_u, neg_i,
           W_user_emb, b_user_emb, W_item_emb, b_item_emb,
           W_pre1_up, W_neigh1_up, W_self1_up,
           W_pre1_pu, W_neigh1_pu, W_self1_pu,
           W_pre2_up, W_neigh2_up, W_self2_up,
           W_pre2_pu, W_neigh2_pu, W_self2_pu):
    hu, hi, m1 = _tc_embed(h_user, h_product, W_user_emb,
                           b_user_emb, W_item_emb, b_item_emb,
                           W_pre1_up, W_pre1_pu)

    def _comb2(a0, a1, b0, b1, apad, bpad, total):
        a = jnp.stack([_pad_idx(a0, apad, total),
                       _pad_idx(a1, apad, total)]).reshape(NC, -1, 8, 128)
        b = jnp.stack([_pad_idx(b0, bpad, total),
                       _pad_idx(b1, bpad, total)]).reshape(NC, -1, 8, 128)
        return jnp.concatenate([a, b], axis=2)

    comb = _comb2(edge_u, edge_i, edge_i, edge_u, 0, N, EPAD)

    cnt = _counts_call(comb)
    s1 = _segsum_call(m1, comb)
    hu1, hi1, m2 = _tc_post_pre(s1, cnt, hu, hi,
                                W_self1_up, W_neigh1_up,
                                W_self1_pu, W_neigh1_pu,
                                W_pre2_up, W_pre2_pu)
    s2 = _segsum_call(m2, comb)
    hu2, hi2, hub, hib = _tc_post_final(s2, cnt, hu1, hi1,
                                        W_self2_up, W_neigh2_up,
                                        W_self2_pu, W_neigh2_pu)

    pu = jnp.stack([_pad_idx(pos_u, 0, PPAD),
                    _pad_idx(neg_u, 0, PPAD)]).reshape(NC, -1, 1, 128)
    pi = jnp.stack([_pad_idx(pos_i, 0, PPAD),
                    _pad_idx(neg_i, 0, PPAD)]).reshape(NC, -1, 1, 128)
    pcomb = jnp.concatenate([pu, pi], axis=2)
    sc = _scores_call(hub, hib, pcomb)
    return hu2, hi2, sc[0, :P], sc[1, :P]


# speedup vs baseline: 1.3354x; 1.0350x over previous
import jax
import jax.numpy as jnp
from jax import lax
from jax.experimental import pallas as pl
from jax.experimental.pallas import tpu as pltpu
from jax.experimental.pallas import tpu_sc as plsc

N = 25000
D = 64
DH = 32
E = 400000
P = 100000
NC = 2
NS = 16

NPAD = 25088
EPAD = 409600
E_PER_TILE = EPAD // NS
ECHUNKS = E_PER_TILE // 512
PPAD = 102400
P_PER_TILE = PPAD // NS
PCHUNKS = P_PER_TILE // 128

ROWS_PER_TILE = NPAD // NS
RB = 5000
NBLK = N // RB


def _dot(a, b):
    return jnp.dot(a, b, preferred_element_type=jnp.float32)


def _l2n(z):
    n = jnp.sqrt(jnp.sum(z * z, axis=1, keepdims=True))
    n = jnp.where(n == 0, jnp.ones_like(n), n)
    return z / n


def _tc_embed_body(huser_ref, hprod_ref, wue_ref, bue_ref, wie_ref, bie_ref,
                   wp_u_ref, wp_i_ref, hu_ref, hi_ref, m_ref):
    hu = _dot(huser_ref[...], wue_ref[...]) + bue_ref[...]
    hi = _dot(hprod_ref[...], wie_ref[...]) + bie_ref[...]
    hu_ref[...] = hu
    hi_ref[...] = hi
    m_ref[0] = jax.nn.relu(_dot(hu, wp_u_ref[...])).astype(jnp.bfloat16)
    m_ref[1] = jax.nn.relu(_dot(hi, wp_i_ref[...])).astype(jnp.bfloat16)


def _tc_embed(h_user, h_product, W_user_emb, b_user_emb, W_item_emb,
              b_item_emb, W_pre1_up, W_pre1_pu):
    full = lambda shape: pl.BlockSpec(shape, lambda i: tuple(0 for _ in shape))
    return pl.pallas_call(
        _tc_embed_body,
        grid=(NBLK,),
        in_specs=[
            pl.BlockSpec((RB, 128), lambda i: (i, 0)),
            pl.BlockSpec((RB, 128), lambda i: (i, 0)),
            full((128, D)), full((1, D)), full((128, D)), full((1, D)),
            full((D, D)), full((D, D)),
        ],
        out_specs=[
            pl.BlockSpec((RB, D), lambda i: (i, 0)),
            pl.BlockSpec((RB, D), lambda i: (i, 0)),
            pl.BlockSpec((2, RB, D), lambda i: (0, i, 0)),
        ],
        out_shape=[
            jax.ShapeDtypeStruct((N, D), jnp.float32),
            jax.ShapeDtypeStruct((N, D), jnp.float32),
            jax.ShapeDtypeStruct((2, N, D), jnp.bfloat16),
        ],
    )(h_user, h_product, W_user_emb, b_user_emb.reshape(1, D), W_item_emb,
      b_item_emb.reshape(1, D), W_pre1_up, W_pre1_pu)


def _make_tc_post(with_pre):
    def body(s_ref, cnt_ref, hu_ref, hi_ref, wsu_ref, wnu_ref,
             wsp_ref, wnp_ref, *rest):
        if with_pre:
            wp2u_ref, wp2p_ref, hu_out, hi_out, m_ref = rest
        else:
            hu_out, hi_out = rest[:2]
        s_i = s_ref[0].astype(jnp.float32)
        s_u = s_ref[1].astype(jnp.float32)
        c_i = cnt_ref[0][:, 0:1]
        c_u = cnt_ref[1][:, 0:1]
        neigh_i = jnp.where(c_i > 0, s_i / jnp.where(c_i > 0, c_i, 1.0), 0.0)
        neigh_u = jnp.where(c_u > 0, s_u / jnp.where(c_u > 0, c_u, 1.0), 0.0)
        hi_n = _l2n(jax.nn.relu(_dot(hi_ref[...], wsu_ref[...])
                                + _dot(neigh_i, wnu_ref[...])))
        hu_n = _l2n(jax.nn.relu(_dot(hu_ref[...], wsp_ref[...])
                                + _dot(neigh_u, wnp_ref[...])))
        hu_out[...] = hu_n
        hi_out[...] = hi_n
        if not with_pre:
            hub_out, hib_out = rest[2:]
            hub_out[...] = hu_n.astype(jnp.bfloat16)
            hib_out[...] = hi_n.astype(jnp.bfloat16)
        if with_pre:
            m_ref[0] = jax.nn.relu(_dot(hu_n, wp2u_ref[...])).astype(jnp.bfloat16)
            m_ref[1] = jax.nn.relu(_dot(hi_n, wp2p_ref[...])).astype(jnp.bfloat16)

    full = lambda shape: pl.BlockSpec(shape, lambda i: tuple(0 for _ in shape))
    in_specs = [
        pl.BlockSpec((2, RB, D), lambda i: (0, i, 0)),
        pl.BlockSpec((2, RB, 16), lambda i: (0, i, 0)),
        pl.BlockSpec((RB, D), lambda i: (i, 0)),
        pl.BlockSpec((RB, D), lambda i: (i, 0)),
        full((D, D)), full((D, D)), full((D, D)), full((D, D)),
    ]
    out_specs = [
        pl.BlockSpec((RB, D), lambda i: (i, 0)),
        pl.BlockSpec((RB, D), lambda i: (i, 0)),
    ]
    out_shape = [
        jax.ShapeDtypeStruct((N, D), jnp.float32),
        jax.ShapeDtypeStruct((N, D), jnp.float32),
    ]
    if with_pre:
        in_specs += [full((D, D)), full((D, D))]
        out_specs += [pl.BlockSpec((2, RB, D), lambda i: (0, i, 0))]
        out_shape += [jax.ShapeDtypeStruct((2, N, D), jnp.bfloat16)]
    else:
        out_specs += [pl.BlockSpec((RB, D), lambda i: (i, 0)),
                      pl.BlockSpec((RB, D), lambda i: (i, 0))]
        out_shape += [jax.ShapeDtypeStruct((N, D), jnp.bfloat16),
                      jax.ShapeDtypeStruct((N, D), jnp.bfloat16)]

    def call(*args):
        return pl.pallas_call(body, grid=(NBLK,), in_specs=in_specs,
                              out_specs=out_specs, out_shape=out_shape)(*args)
    return call


def _zero_rows(ref, nrows, width, dtype=jnp.float32):
    lanes = 32 if dtype == jnp.bfloat16 else 16
    z = jnp.zeros((lanes,), dtype)

    def row(r, _):
        for k in range(width // lanes):
            ref[r, pl.ds(k * lanes, lanes)] = z
        return 0

    lax.fori_loop(0, nrows, row, 0)


def _sc_mesh():
    return plsc.VectorSubcoreMesh(core_axis_name="c", subcore_axis_name="s",
                                  num_cores=NC, num_subcores=NS)


def _sc_params():
    return pltpu.CompilerParams(use_tc_tiling_on_sc=False,
                                needs_layout_passes=False)


def _segsum_body(m_all, comb, s_out, cidx, msg, zbuf, acc, sem0, sem1):
    cid = lax.axis_index("c")
    sid = lax.axis_index("s")

    _zero_rows(zbuf, 112, D, jnp.bfloat16)
    row0 = sid * ROWS_PER_TILE
    g0 = sid * ECHUNKS
    sems = (sem0, sem1)

    def zloop(j, _):
        pltpu.sync_copy(zbuf, acc.at[pl.ds(row0 + j * 112, 112)])
        return 0
    lax.fori_loop(0, ROWS_PER_TILE // 112, zloop, 0)

    plsc.subcore_barrier()

    def fire(b, using):
        for q in range(4):
            pltpu.async_copy(m_all.at[cid].at[cidx.at[using].at[q]],
                             msg.at[b, pl.ds(q * 128, 128)], sems[b])

    def wait(b, using):
        for q in range(4):
            pltpu.make_async_copy(m_all.at[cid].at[cidx.at[using].at[q]],
                                  msg.at[b, pl.ds(q * 128, 128)],
                                  sems[b]).wait()

    def scatter(b, using):
        for q in range(4):
            pltpu.sync_copy(msg.at[b, pl.ds(q * 128, 128)],
                            acc.at[cidx.at[using].at[4 + q]], add=True)

    pltpu.sync_copy(comb.at[cid, pl.ds(g0, 1)], cidx.at[pl.ds(0, 1)])
    fire(0, 0)
    pltpu.sync_copy(comb.at[cid, pl.ds(g0 + 1, 1)], cidx.at[pl.ds(1, 1)])

    def pair(g, _):
        for b in (0, 1):
            j = g * 2 + b
            nb = 1 - b
            wait(b, b)
            fire(nb, nb)
            scatter(b, b)
            jn = lax.min(j + 2, ECHUNKS - 1)
            pltpu.sync_copy(comb.at[cid, pl.ds(g0 + jn, 1)],
                            cidx.at[pl.ds(b, 1)])
        return 0
    lax.fori_loop(0, ECHUNKS // 2, pair, 0)

    wait(0, 0)

    plsc.subcore_barrier()

    def out_loop(j, _):
        rows = pl.ds(row0 + j * 112, 112)
        pltpu.sync_copy(acc.at[rows], zbuf)
        pltpu.sync_copy(zbuf, s_out.at[cid].at[rows])
        return 0
    lax.fori_loop(0, ROWS_PER_TILE // 112, out_loop, 0)


def _segsum_call(m_all, comb):
    return pl.kernel(
        _segsum_body,
        out_type=jax.ShapeDtypeStruct((NC, NPAD, D), jnp.bfloat16),
        mesh=_sc_mesh(),
        compiler_params=_sc_params(),
        scratch_types=[
            pltpu.VMEM((2, 8, 128), jnp.int32),
            pltpu.VMEM((2, 512, D), jnp.bfloat16),
            pltpu.VMEM((112, D), jnp.bfloat16),
            pltpu.VMEM_SHARED((NPAD, D), jnp.bfloat16),
            pltpu.SemaphoreType.DMA,
            pltpu.SemaphoreType.DMA,
        ],
    )(m_all, comb)


def _counts_body(comb, cnt_out, didx, ones_v, cbuf, cacc, sem):
    del sem
    cid = lax.axis_index("c")
    sid = lax.axis_index("s")

    _zero_rows(cbuf, 392, 16)
    one = jnp.ones((16,), jnp.float32)

    def orow(r, _):
        ones_v[r, pl.ds(0, 16)] = one
        return 0
    lax.fori_loop(0, 128, orow, 0)

    row0 = sid * ROWS_PER_TILE

    def czloop(j, _):
        pltpu.sync_copy(cbuf, cacc.at[pl.ds(row0 + j * 392, 392)])
        return 0
    lax.fori_loop(0, ROWS_PER_TILE // 392, czloop, 0)

    plsc.subcore_barrier()

    g0 = sid * ECHUNKS

    def chunk(j, _):
        pltpu.sync_copy(comb.at[cid, g0 + 2 * j, pl.ds(4, 4)],
                        didx.at[pl.ds(0, 4)])
        pltpu.sync_copy(comb.at[cid, g0 + 2 * j + 1, pl.ds(4, 4)],
                        didx.at[pl.ds(4, 4)])
        for q in range(8):
            pltpu.sync_copy(ones_v, cacc.at[didx.at[q]], add=True)
        return 0
    lax.fori_loop(0, ECHUNKS // 2, chunk, 0)

    plsc.subcore_barrier()

    def cout_loop(j, _):
        rows = pl.ds(row0 + j * 392, 392)
        pltpu.sync_copy(cacc.at[rows], cbuf)
        pltpu.sync_copy(cbuf, cnt_out.at[cid].at[rows])
        return 0
    lax.fori_loop(0, ROWS_PER_TILE // 392, cout_loop, 0)


def _counts_call(comb):
    return pl.kernel(
        _counts_body,
        out_type=jax.ShapeDtypeStruct((NC, NPAD, 16), jnp.float32),
        mesh=_sc_mesh(),
        compiler_params=_sc_params(),
        scratch_types=[
            pltpu.VMEM((8, 128), jnp.int32),
            pltpu.VMEM((128, 16), jnp.float32),
            pltpu.VMEM((392, 16), jnp.float32),
            pltpu.VMEM_SHARED((NPAD, 16), jnp.float32),
            pltpu.SemaphoreType.DMA,
        ],
    )(comb)


def _scores_body(hub, hib, pcomb, sc_out,
                 pidx, urows, vrows, tbuf, sbuf, sem0, sem1):
    cid = lax.axis_index("c")
    sid = lax.axis_index("s")
    idx_row0 = sid * (P_PER_TILE // 128)
    lanes = lax.iota(jnp.int32, 16)
    nch = P_PER_TILE // 128
    sems = (sem0, sem1)

    def fire(b, rb):
        pltpu.sync_copy(pcomb.at[cid, pl.ds(rb, 1)], pidx.at[pl.ds(b, 1)])
        pltpu.async_copy(hub.at[pidx.at[b, 0]], urows.at[b], sems[b])
        pltpu.async_copy(hib.at[pidx.at[b, 1]], vrows.at[b], sems[b])

    def wait(b):
        pltpu.make_async_copy(hub.at[pidx.at[b, 0]], urows.at[b],
                              sems[b]).wait()
        pltpu.make_async_copy(hib.at[pidx.at[b, 1]], vrows.at[b],
                              sems[b]).wait()

    def compute(b, rb):
        def group(g, _):
            for p in range(16):
                pa = (urows[b, g * 16 + p, pl.ds(0, 32)]
                      * vrows[b, g * 16 + p, pl.ds(0, 32)]
                      + urows[b, g * 16 + p, pl.ds(32, 32)]
                      * vrows[b, g * 16 + p, pl.ds(32, 32)])
                a0, a1 = plsc.unpack(pa, format=plsc.PackFormat.INTERLEAVED)
                plsc.store_scatter(tbuf, [lanes, jnp.full((16,), p, jnp.int32)],
                                   a0 + a1)
            tot = tbuf[0, pl.ds(0, 16)]
            for rr in range(1, 16):
                tot = tot + tbuf[rr, pl.ds(0, 16)]
            sbuf[pl.ds(g * 16, 16)] = tot
            return 0
        lax.fori_loop(0, 8, group, 0)
        pltpu.sync_copy(sbuf, sc_out.at[cid, pl.ds(rb * 128, 128)])

    fire(0, idx_row0)

    def pair(g, _):
        j0 = g * 2
        for b in (0, 1):
            j = j0 + b
            wait(b)
            jn = lax.min(j + 1, nch - 1)
            fire(1 - b, idx_row0 + jn)
            compute(b, idx_row0 + j)
        return 0
    lax.fori_loop(0, nch // 2, pair, 0)

    wait(0)


def _scores_call(hub, hib, pcomb):
    return pl.kernel(
        _scores_body,
        out_type=jax.ShapeDtypeStruct((NC, PPAD), jnp.float32),
        mesh=_sc_mesh(),
        compiler_params=_sc_params(),
        scratch_types=[
            pltpu.VMEM((2, 2, 128), jnp.int32),
            pltpu.VMEM((2, 128, D), jnp.bfloat16),
            pltpu.VMEM((2, 128, D), jnp.bfloat16),
            pltpu.VMEM((16, 16), jnp.float32),
            pltpu.VMEM((128,), jnp.float32),
            pltpu.SemaphoreType.DMA,
            pltpu.SemaphoreType.DMA,
        ],
    )(hub, hib, pcomb)


def _pad_idx(a, value, total):
    a = a.astype(jnp.int32)
    return jnp.concatenate(
        [a, jnp.full((total - a.shape[0],), value, jnp.int32)])


_tc_post_pre = _make_tc_post(True)
_tc_post_final = _make_tc_post(False)


def kernel(h_user, h_product, edge_u, edge_i, pos_u, pos_i, neg_u, neg_i,
           W_user_emb, b_user_emb, W_item_emb, b_item_emb,
           W_pre1_up, W_neigh1_up, W_self1_up,
           W_pre1_pu, W_neigh1_pu, W_self1_pu,
           W_pre2_up, W_neigh2_up, W_self2_up,
           W_pre2_pu, W_neigh2_pu, W_self2_pu):
    hu, hi, m1 = _tc_embed(h_user, h_product, W_user_emb,
                           b_user_emb, W_item_emb, b_item_emb,
                           W_pre1_up, W_pre1_pu)

    def _comb2(a0, a1, b0, b1, apad, bpad, total):
        a = jnp.stack([_pad_idx(a0, apad, total),
                       _pad_idx(a1, apad, total)]).reshape(NC, -1, 4, 128)
        b = jnp.stack([_pad_idx(b0, bpad, total),
                       _pad_idx(b1, bpad, total)]).reshape(NC, -1, 4, 128)
        return jnp.concatenate([a, b], axis=2)

    comb = _comb2(edge_u, edge_i, edge_i, edge_u, 0, N, EPAD)

    cnt = _counts_call(comb)
    s1 = _segsum_call(m1, comb)
    hu1, hi1, m2 = _tc_post_pre(s1, cnt, hu, hi,
                                W_self1_up, W_neigh1_up,
                                W_self1_pu, W_neigh1_pu,
                                W_pre2_up, W_pre2_pu)
    s2 = _segsum_call(m2, comb)
    hu2, hi2, hub, hib = _tc_post_final(s2, cnt, hu1, hi1,
                                        W_self2_up, W_neigh2_up,
                                        W_self2_pu, W_neigh2_pu)

    pu = jnp.stack([_pad_idx(pos_u, 0, PPAD),
                    _pad_idx(neg_u, 0, PPAD)]).reshape(NC, -1, 1, 128)
    pi = jnp.stack([_pad_idx(pos_i, 0, PPAD),
                    _pad_idx(neg_i, 0, PPAD)]).reshape(NC, -1, 1, 128)
    pcomb = jnp.concatenate([pu, pi], axis=2)
    sc = _scores_call(hub, hib, pcomb)
    return hu2, hi2, sc[0, :P], sc[1, :P]
